# baseline (speedup 1.0000x reference)
import functools

import jax
import jax.numpy as jnp
from jax import lax
from jax.experimental import pallas as pl
from jax.experimental.pallas import tpu as pltpu
from jax.experimental.pallas import tpu_sc as plsc

_NC = 2
_NS = 16
_NW = _NC * _NS
_L = 16

_BATCH = 16384
_COMP = 4
_DIMS = 64

_BPW = _BATCH // _NW
_CHUNK_B = 128
_CHUNK_R = _CHUNK_B * _COMP
_NCH = _BPW // _CHUNK_B


def _sc_body(table_hbm, idx_hbm, w_hbm, out_hbm,
             idx_v, w_v, rows0, rows1, outb0, outb1,
             gsem0, gsem1, osem0, osem1):
    wid = lax.axis_index("s") * _NC + lax.axis_index("c")
    row_base = wid * _BPW * _COMP
    b_base = wid * _BPW

    rows = (rows0, rows1)
    outs = (outb0, outb1)
    gsems = (gsem0, gsem1)
    osems = (osem0, osem1)

    pltpu.sync_copy(idx_hbm.at[pl.ds(row_base, _BPW * _COMP)], idx_v)

    gdesc = [None] * _NCH
    gdesc[0] = pltpu.async_copy(
        table_hbm.at[idx_v.at[pl.ds(0, _CHUNK_R)]], rows[0], gsems[0])

    pltpu.sync_copy(w_hbm.at[pl.ds(row_base, _BPW * _COMP)], w_v)

    odesc = [None] * _NCH
    for c in range(_NCH):
        nxt = c + 1
        if nxt < _NCH:
            gdesc[nxt] = pltpu.async_copy(
                table_hbm.at[idx_v.at[pl.ds(nxt * _CHUNK_R, _CHUNK_R)]],
                rows[nxt % 2], gsems[nxt % 2])
        gdesc[c].wait()
        if c >= 2:
            odesc[c - 2].wait()

        rbuf = rows[c % 2]
        obuf = outs[c % 2]
        woff = c * _CHUNK_R

        def body(g, carry, rbuf=rbuf, obuf=obuf, woff=woff):
            wv = w_v[pl.ds(woff + g * _L, _L)]
            for j in range(_L // _COMP):
                e = g * (_L // _COMP) + j
                rb = e * _COMP
                for s in range(_DIMS // _L):
                    col = pl.ds(s * _L, _L)
                    acc = (rbuf[rb, col] * wv[_COMP * j]
                           + rbuf[rb + 1, col] * wv[_COMP * j + 1]
                           + rbuf[rb + 2, col] * wv[_COMP * j + 2]
                           + rbuf[rb + 3, col] * wv[_COMP * j + 3])
                    obuf[e, col] = acc
            return carry

        lax.fori_loop(0, _CHUNK_B * _COMP // _L, body, 0)

        odesc[c] = pltpu.async_copy(
            obuf, out_hbm.at[pl.ds(b_base + c * _CHUNK_B, _CHUNK_B)],
            osems[c % 2])

    odesc[_NCH - 2].wait()
    odesc[_NCH - 1].wait()


@functools.partial(jax.jit, static_argnames=())
def _sc_lookup(table, idx, w):
    mesh = plsc.VectorSubcoreMesh(core_axis_name="c", subcore_axis_name="s")
    k = pl.kernel(
        _sc_body,
        out_type=jax.ShapeDtypeStruct((_BATCH, _DIMS), jnp.float32),
        mesh=mesh,
        scratch_types=[
            pltpu.VMEM((_BPW * _COMP,), jnp.int32),
            pltpu.VMEM((_BPW * _COMP,), jnp.float32),
            pltpu.VMEM((_CHUNK_R, _DIMS), jnp.float32),
            pltpu.VMEM((_CHUNK_R, _DIMS), jnp.float32),
            pltpu.VMEM((_CHUNK_B, _DIMS), jnp.float32),
            pltpu.VMEM((_CHUNK_B, _DIMS), jnp.float32),
            pltpu.SemaphoreType.DMA,
            pltpu.SemaphoreType.DMA,
            pltpu.SemaphoreType.DMA,
            pltpu.SemaphoreType.DMA,
        ],
        compiler_params=pltpu.CompilerParams(use_tc_tiling_on_sc=False),
    )
    return k(table, idx, w)


def kernel(coords, coord_weights, embeddings):
    t1 = embeddings.shape[1]
    dims = embeddings.shape[-1]
    idx = (coords[..., 0].astype(jnp.int32) * t1
           + coords[..., 1].astype(jnp.int32)).reshape(-1)
    w = coord_weights.reshape(-1)
    table = embeddings.reshape(-1, dims)
    return _sc_lookup(table, idx, w)


# speedup vs baseline: 1.0067x; 1.0067x over previous
import functools

import jax
import jax.numpy as jnp
from jax import lax
from jax.experimental import pallas as pl
from jax.experimental.pallas import tpu as pltpu
from jax.experimental.pallas import tpu_sc as plsc

_NC = 2
_NS = 16
_NW = _NC * _NS
_L = 16

_BATCH = 16384
_COMP = 4
_DIMS = 64
_PAIR = 128

_BPW = _BATCH // _NW
_RPW = _BPW * _COMP
_CHUNK_B = 64
_CHUNK_R = _CHUNK_B * _COMP
_NCH = _BPW // _CHUNK_B


def _sc_body(table_hbm, idx_hbm, w_hbm, out_hbm,
             idx_v, ihi_v, poff_v, w_v, rows0, rows1, outb0, outb1,
             gsem0, gsem1, osem0, osem1):
    wid = lax.axis_index("s") * _NC + lax.axis_index("c")
    row_base = wid * _RPW
    b_base = wid * _BPW

    rows = (rows0, rows1)
    outs = (outb0, outb1)
    gsems = (gsem0, gsem1)
    osems = (osem0, osem1)

    pltpu.sync_copy(idx_hbm.at[pl.ds(row_base, _RPW)], idx_v)

    def split(g, carry):
        sl = pl.ds(g * _L, _L)
        iv = idx_v[sl]
        ihi_v[sl] = lax.shift_right_logical(iv, 1)
        poff_v[sl] = lax.shift_left(jnp.bitwise_and(iv, 1), 6)
        return carry

    lax.fori_loop(0, _RPW // _L, split, 0)

    gdesc = [None] * _NCH
    gdesc[0] = pltpu.async_copy(
        table_hbm.at[ihi_v.at[pl.ds(0, _CHUNK_R)]], rows[0], gsems[0])

    pltpu.sync_copy(w_hbm.at[pl.ds(row_base, _RPW)], w_v)

    odesc = [None] * _NCH
    for c in range(_NCH):
        nxt = c + 1
        if nxt < _NCH:
            gdesc[nxt] = pltpu.async_copy(
                table_hbm.at[ihi_v.at[pl.ds(nxt * _CHUNK_R, _CHUNK_R)]],
                rows[nxt % 2], gsems[nxt % 2])
        gdesc[c].wait()
        if c >= 2:
            odesc[c - 2].wait()

        rbuf = rows[c % 2]
        obuf = outs[c % 2]
        woff = c * _CHUNK_R

        def body(g, carry, rbuf=rbuf, obuf=obuf, woff=woff):
            wsl = pl.ds(woff + g * _L, _L)
            wv = w_v[wsl]
            pv = poff_v[wsl]
            for j in range(_L // _COMP):
                e = g * (_L // _COMP) + j
                rb = e * _COMP
                o0 = pv[_COMP * j]
                o1 = pv[_COMP * j + 1]
                o2 = pv[_COMP * j + 2]
                o3 = pv[_COMP * j + 3]
                w0 = wv[_COMP * j]
                w1 = wv[_COMP * j + 1]
                w2 = wv[_COMP * j + 2]
                w3 = wv[_COMP * j + 3]
                for s in range(_DIMS // _L):
                    acc = (rbuf[rb, pl.ds(o0 + s * _L, _L)] * w0
                           + rbuf[rb + 1, pl.ds(o1 + s * _L, _L)] * w1
                           + rbuf[rb + 2, pl.ds(o2 + s * _L, _L)] * w2
                           + rbuf[rb + 3, pl.ds(o3 + s * _L, _L)] * w3)
                    obuf[e, pl.ds(s * _L, _L)] = acc
            return carry

        lax.fori_loop(0, _CHUNK_R // _L, body, 0)

        odesc[c] = pltpu.async_copy(
            obuf, out_hbm.at[pl.ds(b_base + c * _CHUNK_B, _CHUNK_B)],
            osems[c % 2])

    odesc[_NCH - 2].wait()
    odesc[_NCH - 1].wait()


@jax.jit
def _sc_lookup(table, idx, w):
    mesh = plsc.VectorSubcoreMesh(core_axis_name="c", subcore_axis_name="s")
    k = pl.kernel(
        _sc_body,
        out_type=jax.ShapeDtypeStruct((_BATCH, _DIMS), jnp.float32),
        mesh=mesh,
        scratch_types=[
            pltpu.VMEM((_RPW,), jnp.int32),
            pltpu.VMEM((_RPW,), jnp.int32),
            pltpu.VMEM((_RPW,), jnp.int32),
            pltpu.VMEM((_RPW,), jnp.float32),
            pltpu.VMEM((_CHUNK_R, _PAIR), jnp.float32),
            pltpu.VMEM((_CHUNK_R, _PAIR), jnp.float32),
            pltpu.VMEM((_CHUNK_B, _DIMS), jnp.float32),
            pltpu.VMEM((_CHUNK_B, _DIMS), jnp.float32),
            pltpu.SemaphoreType.DMA,
            pltpu.SemaphoreType.DMA,
            pltpu.SemaphoreType.DMA,
            pltpu.SemaphoreType.DMA,
        ],
    )
    return k(table, idx, w)


def kernel(coords, coord_weights, embeddings):
    t1 = embeddings.shape[1]
    dims = embeddings.shape[-1]
    idx = (coords[..., 0].astype(jnp.int32) * t1
           + coords[..., 1].astype(jnp.int32)).reshape(-1)
    w = coord_weights.reshape(-1)
    table = embeddings.reshape(-1, 2 * dims)
    return _sc_lookup(table, idx, w)


# speedup vs baseline: 1.0092x; 1.0025x over previous
import jax
import jax.numpy as jnp
from jax import lax
from jax.experimental import pallas as pl
from jax.experimental.pallas import tpu as pltpu
from jax.experimental.pallas import tpu_sc as plsc

_NC = 2
_NS = 16
_NW = _NC * _NS
_L = 16

_BATCH = 16384
_COMP = 4
_DIMS = 64
_PAIR = 128

_BPW = _BATCH // _NW
_RPW = _BPW * _COMP
_CHUNK_R = 128
_CHUNK_B = _CHUNK_R // _COMP
_NCH = _RPW // _CHUNK_R
_RING = 4


def _sc_body(table_hbm, idx_hbm, w_hbm, out_hbm,
             idx_v, ihi2, poff_v, w_v,
             rows0, rows1, rows2, rows3,
             outb0, outb1, outb2, outb3,
             gsem0, gsem1, gsem2, gsem3,
             osem0, osem1, osem2, osem3):
    wid = lax.axis_index("s") * _NC + lax.axis_index("c")
    row_base = wid * _RPW
    b_base = wid * _BPW

    rows = (rows0, rows1, rows2, rows3)
    outs = (outb0, outb1, outb2, outb3)
    gsems = (gsem0, gsem1, gsem2, gsem3)
    osems = (osem0, osem1, osem2, osem3)

    pltpu.sync_copy(idx_hbm.at[pl.ds(row_base, _RPW)], idx_v)

    def split(g, carry):
        sl = pl.ds(g * _L, _L)
        iv = idx_v[sl]
        poff_v[sl] = lax.shift_left(jnp.bitwise_and(iv, 1), 6)
        return carry

    lax.fori_loop(0, _RPW // _L, split, 0)

    def split2(c2, carry):
        def inner(l, carry2):
            ihi2[c2, pl.ds(l * _L, _L)] = lax.shift_right_logical(
                idx_v[pl.ds(c2 * _CHUNK_R + l * _L, _L)], 1)
            return carry2
        return lax.fori_loop(0, _CHUNK_R // _L, inner, carry)

    lax.fori_loop(0, _NCH, split2, 0)

    gdesc = [None] * _NCH
    for p in range(_RING - 1):
        gdesc[p] = pltpu.async_copy(
            table_hbm.at[ihi2.at[p]], rows[p], gsems[p])

    pltpu.sync_copy(w_hbm.at[pl.ds(row_base, _RPW)], w_v)

    odesc = [None] * _NCH
    for c in range(_NCH):
        nxt = c + _RING - 1
        if nxt < _NCH:
            gdesc[nxt] = pltpu.async_copy(
                table_hbm.at[ihi2.at[nxt]], rows[nxt % _RING],
                gsems[nxt % _RING])
        gdesc[c].wait()
        if c >= _RING:
            odesc[c - _RING].wait()

        rbuf = rows[c % _RING]
        obuf = outs[c % _RING]
        woff = c * _CHUNK_R

        def body(g, carry, rbuf=rbuf, obuf=obuf, woff=woff):
            wsl = pl.ds(woff + g * _L, _L)
            wv = w_v[wsl]
            pv = poff_v[wsl]
            for j in range(_L // _COMP):
                e = g * (_L // _COMP) + j
                rb = e * _COMP
                o0 = pv[_COMP * j]
                o1 = pv[_COMP * j + 1]
                o2 = pv[_COMP * j + 2]
                o3 = pv[_COMP * j + 3]
                w0 = wv[_COMP * j]
                w1 = wv[_COMP * j + 1]
                w2 = wv[_COMP * j + 2]
                w3 = wv[_COMP * j + 3]
                for s in range(_DIMS // _L):
                    acc = (rbuf[rb, pl.ds(o0 + s * _L, _L)] * w0
                           + rbuf[rb + 1, pl.ds(o1 + s * _L, _L)] * w1
                           + rbuf[rb + 2, pl.ds(o2 + s * _L, _L)] * w2
                           + rbuf[rb + 3, pl.ds(o3 + s * _L, _L)] * w3)
                    obuf[e, pl.ds(s * _L, _L)] = acc
            return carry

        lax.fori_loop(0, _CHUNK_R // _L, body, 0)

        odesc[c] = pltpu.async_copy(
            obuf, out_hbm.at[pl.ds(b_base + c * _CHUNK_B, _CHUNK_B)],
            osems[c % _RING])

    for c in range(_NCH - _RING, _NCH):
        odesc[c].wait()


@jax.jit
def _sc_lookup(table, idx, w):
    mesh = plsc.VectorSubcoreMesh(core_axis_name="c", subcore_axis_name="s")
    k = pl.kernel(
        _sc_body,
        out_type=jax.ShapeDtypeStruct((_BATCH, _DIMS), jnp.float32),
        mesh=mesh,
        scratch_types=[
            pltpu.VMEM((_RPW,), jnp.int32),
            pltpu.VMEM((_NCH, _CHUNK_R), jnp.int32),
            pltpu.VMEM((_RPW,), jnp.int32),
            pltpu.VMEM((_RPW,), jnp.float32),
            pltpu.VMEM((_CHUNK_R, _PAIR), jnp.float32),
            pltpu.VMEM((_CHUNK_R, _PAIR), jnp.float32),
            pltpu.VMEM((_CHUNK_R, _PAIR), jnp.float32),
            pltpu.VMEM((_CHUNK_R, _PAIR), jnp.float32),
            pltpu.VMEM((_CHUNK_B, _DIMS), jnp.float32),
            pltpu.VMEM((_CHUNK_B, _DIMS), jnp.float32),
            pltpu.VMEM((_CHUNK_B, _DIMS), jnp.float32),
            pltpu.VMEM((_CHUNK_B, _DIMS), jnp.float32),
            pltpu.SemaphoreType.DMA,
            pltpu.SemaphoreType.DMA,
            pltpu.SemaphoreType.DMA,
            pltpu.SemaphoreType.DMA,
            pltpu.SemaphoreType.DMA,
            pltpu.SemaphoreType.DMA,
            pltpu.SemaphoreType.DMA,
            pltpu.SemaphoreType.DMA,
        ],
    )
    return k(table, idx, w)


def kernel(coords, coord_weights, embeddings):
    t1 = embeddings.shape[1]
    dims = embeddings.shape[-1]
    idx = (coords[..., 0].astype(jnp.int32) * t1
           + coords[..., 1].astype(jnp.int32)).reshape(-1)
    w = coord_weights.reshape(-1)
    table = embeddings.reshape(-1, 2 * dims)
    return _sc_lookup(table, idx, w)
